# baseline (speedup 1.0000x reference)
import functools
import math

import jax
import jax.numpy as jnp
from jax.experimental import pallas as pl
from jax.experimental.pallas import tpu as pltpu
from jax.experimental.pallas import tpu_sc as plsc

_TARGET_SNR = 0.3
_EPS = 1e-12
_ROW_ITERS = 16
_TAU_ITERS = 20


def _stage1_body(x_ref, mask_ref, qn_ref, rms_ref, li_ref):
    _, S, D = x_ref.shape
    xb = x_ref[0]
    mb = mask_ref[0]
    iota_s = jax.lax.broadcasted_iota(jnp.int32, (1, S), 1)
    cnt = jnp.sum(mb, axis=1, keepdims=True)
    denom = jnp.maximum(cnt, 1.0)
    li_b = (jnp.maximum(cnt, 1.0) - 1.0).astype(jnp.int32)
    qsum = jax.lax.dot_general(
        mb, xb, (((1,), (0,)), ((), ())),
        preferred_element_type=jnp.float32)
    qmean = qsum / denom
    qn = qmean / jnp.maximum(
        jnp.sqrt(jnp.sum(qmean * qmean, axis=1, keepdims=True)), _EPS)
    qn_ref[...] = qn.astype(jnp.bfloat16)[None]
    onehot = (iota_s == li_b).astype(jnp.float32)
    host = jax.lax.dot_general(
        onehot, xb, (((1,), (0,)), ((), ())),
        preferred_element_type=jnp.float32)
    rms = jnp.sqrt(jnp.mean(host * host, axis=1, keepdims=True))
    rms_ref[...] = jnp.broadcast_to(rms, (1, 128))[None]
    li_ref[...] = jnp.broadcast_to(li_b, (1, 128))[None]


def _stage2_body(k_ref, q_ref, kn_ref, kn8_ref, sims_ref):
    k = k_ref[...]
    kn = k / jnp.maximum(jnp.sqrt(jnp.sum(k * k, axis=1, keepdims=True)), _EPS)
    knb = kn.astype(jnp.bfloat16)
    kn_ref[...] = knb
    kn8_ref[...] = kn.astype(jnp.float8_e4m3fn)
    sims_ref[...] = jax.lax.dot_general(
        q_ref[...], knb, (((1,), (1,)), ((), ())),
        preferred_element_type=jnp.float32)


def _sc_sims_stats(sims, B, NK):
    nch = NK // 128
    mesh = plsc.VectorSubcoreMesh(core_axis_name="c", subcore_axis_name="s")
    out_t = [jax.ShapeDtypeStruct((B, nch, 16), jnp.float32)] * 3

    @functools.partial(pl.kernel, out_type=out_t, mesh=mesh, scratch_types=[])
    def run(sims_hbm, mx_hbm, sm_hbm, sq_hbm):
        def body(in_vmem, mx_vmem, sm_vmem, sq_vmem):
            v0 = in_vmem.at[0, pl.ds(0, 16)][...]
            mx, sm, sq = v0, v0, v0 * v0
            for c in range(1, 8):
                v = in_vmem.at[0, pl.ds(16 * c, 16)][...]
                mx = jnp.maximum(mx, v)
                sm = sm + v
                sq = sq + v * v
            mx_vmem.at[0, 0, pl.ds(0, 16)][...] = mx
            sm_vmem.at[0, 0, pl.ds(0, 16)][...] = sm
            sq_vmem.at[0, 0, pl.ds(0, 16)][...] = sq

        pltpu.emit_pipeline(
            body,
            grid=(B, nch),
            in_specs=[pl.BlockSpec((1, 128), lambda b, c: (b, c))],
            out_specs=[
                pl.BlockSpec((1, 1, 16), lambda b, c: (b, c, 0)),
                pl.BlockSpec((1, 1, 16), lambda b, c: (b, c, 0)),
                pl.BlockSpec((1, 1, 16), lambda b, c: (b, c, 0)),
            ],
            core_axis_name=("c", "s"),
            dimension_semantics=(pltpu.PARALLEL, pltpu.PARALLEL),
        )(sims_hbm, mx_hbm, sm_hbm, sq_hbm)

    return run(sims)


def _stage3_body(il_ref, jl_ref, ki_ref, kj_ref, eye_ref,
                 nmrow_ref, nmcol_ref,
                 rpart_ref, cpart_ref, *, nkb, t_total, strip):
    t = pl.program_id(0)
    i = il_ref[t]
    j = jl_ref[t]
    ki = ki_ref[...]
    KB = ki.shape[0]
    first_of_i = i == j

    @pl.when(t == 0)
    def _init_cpart():
        cpart_ref[...] = jnp.full((nkb, KB), -jnp.inf, jnp.float32)

    rows8 = jax.lax.broadcasted_iota(jnp.int32, (nkb, 1), 0)
    rows = jax.lax.broadcasted_iota(jnp.int32, (KB, strip), 0)
    cols = jax.lax.broadcasted_iota(jnp.int32, (KB, strip), 1)

    rp = None
    cms = []
    for s in range(KB // strip):
        gs = jax.lax.dot_general(
            ki, kj_ref[pl.ds(s * strip, strip), :],
            (((1,), (1,)), ((), ())),
            preferred_element_type=jnp.float32)
        on_diag = jnp.logical_and(rows == cols + s * strip, i == j)
        gs = jnp.where(on_diag, -jnp.inf, gs)
        rps = gs[:, 0:128]
        for kk in range(1, strip // 128):
            rps = jnp.maximum(rps, gs[:, kk * 128:(kk + 1) * 128])
        rp = rps if rp is None else jnp.maximum(rp, rps)
        cms.append(jnp.max(gs, axis=0, keepdims=True))

    rpart_ref[...] = jnp.where(first_of_i, rp,
                               jnp.maximum(rpart_ref[...], rp))

    @pl.when(j == nkb - 1)
    def _emit_row():
        rpT = jax.lax.dot_general(
            rpart_ref[...], eye_ref[...], (((0,), (0,)), ((), ())),
            preferred_element_type=jnp.float32)
        nmrow_ref[...] = jnp.max(rpT, axis=0, keepdims=True)[None]

    cm = jnp.concatenate(cms, axis=1)
    upd = jnp.where(rows8 == j, jnp.broadcast_to(cm, (nkb, KB)), -jnp.inf)
    cpart_ref[...] = jnp.maximum(cpart_ref[...], upd)

    @pl.when(t == t_total - 1)
    def _emit_col():
        nmcol_ref[...] = cpart_ref[...]


def _stage4_body(sims_ref, nma_ref, nmb_ref, rms_ref, pmx_ref, psm_ref,
                 psq_ref, w_ref, gs_ref, *, nk, d, eff_k, temperature):
    sims = sims_ref[...]
    nm = jnp.maximum(nma_ref[...], nmb_ref[...])
    sqrt_d = math.sqrt(float(d))

    mean_raw = jnp.sum(psm_ref[...], axis=1, keepdims=True) / float(nk)
    ex2 = jnp.sum(psq_ref[...], axis=1, keepdims=True) / float(nk)
    var_raw = jnp.maximum(ex2 - mean_raw * mean_raw, 0.0)
    std_raw = jnp.maximum(jnp.sqrt(var_raw), 1e-6)
    max_raw = jnp.max(pmx_ref[...], axis=1, keepdims=True)
    z_peak = (max_raw - mean_raw) / std_raw
    gate_peak = jax.nn.sigmoid(z_peak * sqrt_d)

    rmin_s = jnp.min(sims, axis=1, keepdims=True)
    lo_s = rmin_s - 1.0
    hi_s = max_raw + 1.0
    nm_min = jnp.min(nm)
    nm_max = jnp.max(nm)
    k1 = float((nk + 1) // 2)
    k2 = float(nk // 2 + 1)
    lo_a = lo_b = nm_min - 1.0
    hi_a = hi_b = nm_max + 1.0
    for it in range(max(_ROW_ITERS, _TAU_ITERS)):
        if it < _ROW_ITERS:
            mid_s = 0.5 * (lo_s + hi_s)
            cnt_s = jnp.sum((sims >= mid_s).astype(jnp.float32),
                            axis=1, keepdims=True)
            ge_s = cnt_s >= float(eff_k)
            lo_s = jnp.where(ge_s, mid_s, lo_s)
            hi_s = jnp.where(ge_s, hi_s, mid_s)
        if it < _TAU_ITERS:
            mid_a = 0.5 * (lo_a + hi_a)
            cnt_a = jnp.sum((nm >= mid_a).astype(jnp.float32))
            ge_a = cnt_a >= k1
            lo_a = jnp.where(ge_a, mid_a, lo_a)
            hi_a = jnp.where(ge_a, hi_a, mid_a)
            mid_b = 0.5 * (lo_b + hi_b)
            cnt_b = jnp.sum((nm >= mid_b).astype(jnp.float32))
            ge_b = cnt_b >= k2
            lo_b = jnp.where(ge_b, mid_b, lo_b)
            hi_b = jnp.where(ge_b, hi_b, mid_b)
    thr = jnp.max(jnp.where(sims < hi_s, sims, rmin_s - 2.0),
                  axis=1, keepdims=True)
    va = jnp.max(jnp.where(nm < hi_a, nm, nm_min - 2.0))
    vb = jnp.max(jnp.where(nm < hi_b, nm, nm_min - 2.0))
    tau = 0.5 * (va + vb)

    mu_nm = jnp.mean(nm)
    sigma = jnp.maximum(jnp.sqrt(jnp.mean((nm - mu_nm) ** 2)), 1e-6)
    gate_manifold = jax.nn.sigmoid((max_raw - tau) / sigma * sqrt_d)

    if eff_k < nk:
        logits = jnp.where(sims >= thr, sims, -1e9)
    else:
        logits = sims
    ex = jnp.exp((logits - max_raw) / temperature)
    w_ref[...] = ex / jnp.sum(ex, axis=1, keepdims=True)

    gate = gate_peak * gate_manifold
    magnitude = rms_ref[...] * _TARGET_SNR
    gs_ref[...] = gate * magnitude


def _stage5_body(w_ref, v_ref, acc_ref):
    j = pl.program_id(0)

    @pl.when(j == 0)
    def _init():
        acc_ref[...] = jnp.zeros_like(acc_ref)

    part = jax.lax.dot_general(
        w_ref[...].astype(jnp.bfloat16), v_ref[...].astype(jnp.bfloat16),
        (((1,), (0,)), ((), ())), preferred_element_type=jnp.float32)
    acc_ref[...] = acc_ref[...] + part


def _stage6_body(x_ref, ret_ref, gs_ref, li_ref, out_ref):
    _, S, D = x_ref.shape
    r = ret_ref[0]
    norm = jnp.sqrt(jnp.sum(r * r, axis=1, keepdims=True))
    delta = gs_ref[0, 0, 0] * (r / jnp.maximum(norm, _EPS))
    iota_col = jax.lax.broadcasted_iota(jnp.int32, (S, 1), 0)
    sel = (iota_col == li_ref[0, 0, 0]).astype(jnp.float32)
    out_ref[0] = x_ref[0] + sel * delta


def kernel(x, attention_mask, keys, values):
    B, S, D = x.shape
    NK = keys.shape[0]
    temperature = math.sqrt(math.log1p(float(NK))) / max(float(D), 1.0)
    eff_k = min(NK, max(2, int(math.ceil(
        math.sqrt(float(NK)) * math.log1p(float(NK))))))

    mask3 = attention_mask.astype(jnp.float32).reshape(B, 1, S)

    qn3, rms3, li3 = pl.pallas_call(
        _stage1_body,
        grid=(B,),
        in_specs=[
            pl.BlockSpec((1, S, D), lambda b: (b, 0, 0)),
            pl.BlockSpec((1, 1, S), lambda b: (b, 0, 0)),
        ],
        out_specs=[
            pl.BlockSpec((1, 1, D), lambda b: (b, 0, 0)),
            pl.BlockSpec((1, 1, 128), lambda b: (b, 0, 0)),
            pl.BlockSpec((1, 1, 128), lambda b: (b, 0, 0)),
        ],
        out_shape=[
            jax.ShapeDtypeStruct((B, 1, D), jnp.bfloat16),
            jax.ShapeDtypeStruct((B, 1, 128), jnp.float32),
            jax.ShapeDtypeStruct((B, 1, 128), jnp.int32),
        ],
    )(x, mask3)
    qn = qn3.reshape(B, D)
    rmsarr = rms3.reshape(B, 128)

    KB = 1024
    nkb = NK // KB
    kn, kn8, sims = pl.pallas_call(
        _stage2_body,
        grid=(nkb,),
        in_specs=[
            pl.BlockSpec((KB, D), lambda i: (i, 0)),
            pl.BlockSpec((B, D), lambda i: (0, 0)),
        ],
        out_specs=[
            pl.BlockSpec((KB, D), lambda i: (i, 0)),
            pl.BlockSpec((KB, D), lambda i: (i, 0)),
            pl.BlockSpec((B, KB), lambda i: (0, i)),
        ],
        out_shape=[
            jax.ShapeDtypeStruct((NK, D), jnp.bfloat16),
            jax.ShapeDtypeStruct((NK, D), jnp.float8_e4m3fn),
            jax.ShapeDtypeStruct((B, NK), jnp.float32),
        ],
    )(keys, qn)

    pairs = [(i, j) for i in range(nkb) for j in range(i, nkb)]
    t_total = len(pairs)
    il = jnp.asarray([p[0] for p in pairs], dtype=jnp.int32)
    jl = jnp.asarray([p[1] for p in pairs], dtype=jnp.int32)
    nmrow, nmcol = pl.pallas_call(
        functools.partial(_stage3_body, nkb=nkb, t_total=t_total, strip=256),
        grid_spec=pltpu.PrefetchScalarGridSpec(
            num_scalar_prefetch=2,
            grid=(t_total,),
            in_specs=[
                pl.BlockSpec((KB, D), lambda t, il_r, jl_r: (il_r[t], 0)),
                pl.BlockSpec((KB, D), lambda t, il_r, jl_r: (jl_r[t], 0)),
                pl.BlockSpec((KB, KB), lambda t, il_r, jl_r: (0, 0)),
            ],
            out_specs=[
                pl.BlockSpec((1, 1, KB), lambda t, il_r, jl_r: (il_r[t], 0, 0)),
                pl.BlockSpec((nkb, KB), lambda t, il_r, jl_r: (0, 0)),
            ],
            scratch_shapes=[
                pltpu.VMEM((KB, 128), jnp.float32),
                pltpu.VMEM((nkb, KB), jnp.float32),
            ],
        ),
        out_shape=[
            jax.ShapeDtypeStruct((nkb, 1, KB), jnp.float32),
            jax.ShapeDtypeStruct((nkb, KB), jnp.float32),
        ],
    )(il, jl, kn8, kn8, jnp.eye(KB, dtype=jnp.float32))

    nma = nmrow.reshape(NK // 128, 128)
    nmb = nmcol.reshape(NK // 128, 128)
    pmx, psm, psq = _sc_sims_stats(sims, B, NK)
    pmx2 = pmx.reshape(B, NK // 8)
    psm2 = psm.reshape(B, NK // 8)
    psq2 = psq.reshape(B, NK // 8)
    weights, gscale = pl.pallas_call(
        functools.partial(_stage4_body, nk=NK, d=D, eff_k=eff_k,
                          temperature=temperature),
        out_shape=[
            jax.ShapeDtypeStruct((B, NK), jnp.float32),
            jax.ShapeDtypeStruct((B, 128), jnp.float32),
        ],
    )(sims, nma, nmb, rmsarr, pmx2, psm2, psq2)

    retrieved = pl.pallas_call(
        _stage5_body,
        grid=(nkb,),
        in_specs=[
            pl.BlockSpec((B, KB), lambda j: (0, j)),
            pl.BlockSpec((KB, D), lambda j: (j, 0)),
        ],
        out_specs=pl.BlockSpec((B, D), lambda j: (0, 0)),
        out_shape=jax.ShapeDtypeStruct((B, D), jnp.float32),
    )(weights, values)

    ret3 = retrieved.reshape(B, 1, D)
    gs3 = gscale.reshape(B, 1, 128)
    out = pl.pallas_call(
        _stage6_body,
        grid=(B,),
        in_specs=[
            pl.BlockSpec((1, S, D), lambda b: (b, 0, 0)),
            pl.BlockSpec((1, 1, D), lambda b: (b, 0, 0)),
            pl.BlockSpec((1, 1, 128), lambda b: (b, 0, 0)),
            pl.BlockSpec((1, 1, 128), lambda b: (b, 0, 0)),
        ],
        out_specs=pl.BlockSpec((1, S, D), lambda b: (b, 0, 0)),
        out_shape=jax.ShapeDtypeStruct((B, S, D), jnp.float32),
    )(x, ret3, gs3, li3)

    return out


# speedup vs baseline: 1.1394x; 1.1394x over previous
import functools
import math

import jax
import jax.numpy as jnp
from jax.experimental import pallas as pl
from jax.experimental.pallas import tpu as pltpu

_TARGET_SNR = 0.3
_EPS = 1e-12
_ROW_ITERS = 16
_TAU_ITERS = 20


def _stage1_body(x_ref, mask_ref, qn_ref, rms_ref, li_ref):
    _, S, D = x_ref.shape
    xb = x_ref[0]
    mb = mask_ref[0]
    iota_s = jax.lax.broadcasted_iota(jnp.int32, (1, S), 1)
    cnt = jnp.sum(mb, axis=1, keepdims=True)
    denom = jnp.maximum(cnt, 1.0)
    li_b = (jnp.maximum(cnt, 1.0) - 1.0).astype(jnp.int32)
    qsum = jax.lax.dot_general(
        mb, xb, (((1,), (0,)), ((), ())),
        preferred_element_type=jnp.float32)
    qmean = qsum / denom
    qn = qmean / jnp.maximum(
        jnp.sqrt(jnp.sum(qmean * qmean, axis=1, keepdims=True)), _EPS)
    qn_ref[...] = qn.astype(jnp.bfloat16)[None]
    onehot = (iota_s == li_b).astype(jnp.float32)
    host = jax.lax.dot_general(
        onehot, xb, (((1,), (0,)), ((), ())),
        preferred_element_type=jnp.float32)
    rms = jnp.sqrt(jnp.mean(host * host, axis=1, keepdims=True))
    rms_ref[...] = jnp.broadcast_to(rms, (1, 128))[None]
    li_ref[...] = jnp.broadcast_to(li_b, (1, 128))[None]


def _stage2_body(k_ref, q_ref, kn_ref, kn8_ref, sims_ref):
    k = k_ref[...]
    kn = k / jnp.maximum(jnp.sqrt(jnp.sum(k * k, axis=1, keepdims=True)), _EPS)
    knb = kn.astype(jnp.bfloat16)
    kn_ref[...] = knb
    kn8_ref[...] = kn.astype(jnp.float8_e4m3fn)
    sims_ref[...] = jax.lax.dot_general(
        q_ref[...], knb, (((1,), (1,)), ((), ())),
        preferred_element_type=jnp.float32)


def _stage3_body(il_ref, jl_ref, ki_ref, kj_ref, eye_ref,
                 nmrow_ref, nmcol_ref,
                 rpart_ref, cpart_ref, *, nkb, t_total, strip):
    t = pl.program_id(0)
    i = il_ref[t]
    j = jl_ref[t]
    ki = ki_ref[...]
    KB = ki.shape[0]
    first_of_i = i == j

    @pl.when(t == 0)
    def _init_cpart():
        cpart_ref[...] = jnp.full((nkb, KB), -jnp.inf, jnp.float32)

    rows8 = jax.lax.broadcasted_iota(jnp.int32, (nkb, 1), 0)
    rows = jax.lax.broadcasted_iota(jnp.int32, (KB, strip), 0)
    cols = jax.lax.broadcasted_iota(jnp.int32, (KB, strip), 1)

    rp = None
    cms = []
    for s in range(KB // strip):
        gs = jax.lax.dot_general(
            ki, kj_ref[pl.ds(s * strip, strip), :],
            (((1,), (1,)), ((), ())),
            preferred_element_type=jnp.float32)
        on_diag = jnp.logical_and(rows == cols + s * strip, i == j)
        gs = jnp.where(on_diag, -jnp.inf, gs)
        rps = gs[:, 0:128]
        for kk in range(1, strip // 128):
            rps = jnp.maximum(rps, gs[:, kk * 128:(kk + 1) * 128])
        rp = rps if rp is None else jnp.maximum(rp, rps)
        cms.append(jnp.max(gs, axis=0, keepdims=True))

    rpart_ref[...] = jnp.where(first_of_i, rp,
                               jnp.maximum(rpart_ref[...], rp))

    @pl.when(j == nkb - 1)
    def _emit_row():
        rpT = jax.lax.dot_general(
            rpart_ref[...], eye_ref[...], (((0,), (0,)), ((), ())),
            preferred_element_type=jnp.float32)
        nmrow_ref[...] = jnp.max(rpT, axis=0, keepdims=True)[None]

    cm = jnp.concatenate(cms, axis=1)
    upd = jnp.where(rows8 == j, jnp.broadcast_to(cm, (nkb, KB)), -jnp.inf)
    cpart_ref[...] = jnp.maximum(cpart_ref[...], upd)

    @pl.when(t == t_total - 1)
    def _emit_col():
        nmcol_ref[...] = cpart_ref[...]


def _stage4_body(sims_ref, nma_ref, nmb_ref, rms_ref, w_ref, gs_ref, *,
                 nk, d, eff_k, temperature):
    sims = sims_ref[...]
    nm = jnp.maximum(nma_ref[...], nmb_ref[...])
    sqrt_d = math.sqrt(float(d))

    mean_raw = jnp.mean(sims, axis=1, keepdims=True)
    var_raw = jnp.mean((sims - mean_raw) ** 2, axis=1, keepdims=True)
    std_raw = jnp.maximum(jnp.sqrt(var_raw), 1e-6)
    max_raw = jnp.max(sims, axis=1, keepdims=True)
    z_peak = (max_raw - mean_raw) / std_raw
    gate_peak = jax.nn.sigmoid(z_peak * sqrt_d)

    rmin_s = jnp.min(sims, axis=1, keepdims=True)
    lo_s = rmin_s - 1.0
    hi_s = max_raw + 1.0
    nm_min = jnp.min(nm)
    nm_max = jnp.max(nm)
    k1 = float((nk + 1) // 2)
    k2 = float(nk // 2 + 1)
    lo_a = lo_b = nm_min - 1.0
    hi_a = hi_b = nm_max + 1.0
    for it in range(max(_ROW_ITERS, _TAU_ITERS)):
        if it < _ROW_ITERS:
            mid_s = 0.5 * (lo_s + hi_s)
            cnt_s = jnp.sum((sims >= mid_s).astype(jnp.float32),
                            axis=1, keepdims=True)
            ge_s = cnt_s >= float(eff_k)
            lo_s = jnp.where(ge_s, mid_s, lo_s)
            hi_s = jnp.where(ge_s, hi_s, mid_s)
        if it < _TAU_ITERS:
            mid_a = 0.5 * (lo_a + hi_a)
            cnt_a = jnp.sum((nm >= mid_a).astype(jnp.float32))
            ge_a = cnt_a >= k1
            lo_a = jnp.where(ge_a, mid_a, lo_a)
            hi_a = jnp.where(ge_a, hi_a, mid_a)
            mid_b = 0.5 * (lo_b + hi_b)
            cnt_b = jnp.sum((nm >= mid_b).astype(jnp.float32))
            ge_b = cnt_b >= k2
            lo_b = jnp.where(ge_b, mid_b, lo_b)
            hi_b = jnp.where(ge_b, hi_b, mid_b)
    thr = jnp.max(jnp.where(sims < hi_s, sims, rmin_s - 2.0),
                  axis=1, keepdims=True)
    va = jnp.max(jnp.where(nm < hi_a, nm, nm_min - 2.0))
    vb = jnp.max(jnp.where(nm < hi_b, nm, nm_min - 2.0))
    tau = 0.5 * (va + vb)

    mu_nm = jnp.mean(nm)
    sigma = jnp.maximum(jnp.sqrt(jnp.mean((nm - mu_nm) ** 2)), 1e-6)
    gate_manifold = jax.nn.sigmoid((max_raw - tau) / sigma * sqrt_d)

    if eff_k < nk:
        logits = jnp.where(sims >= thr, sims, -1e9)
    else:
        logits = sims
    ex = jnp.exp((logits - max_raw) / temperature)
    w_ref[...] = ex / jnp.sum(ex, axis=1, keepdims=True)

    gate = gate_peak * gate_manifold
    magnitude = rms_ref[...] * _TARGET_SNR
    gs_ref[...] = gate * magnitude


def _stage5_body(w_ref, v_ref, acc_ref):
    j = pl.program_id(0)

    @pl.when(j == 0)
    def _init():
        acc_ref[...] = jnp.zeros_like(acc_ref)

    part = jax.lax.dot_general(
        w_ref[...].astype(jnp.bfloat16), v_ref[...].astype(jnp.bfloat16),
        (((1,), (0,)), ((), ())), preferred_element_type=jnp.float32)
    acc_ref[...] = acc_ref[...] + part


def _stage6_body(x_ref, ret_ref, gs_ref, li_ref, out_ref):
    _, S, D = x_ref.shape
    r = ret_ref[0]
    norm = jnp.sqrt(jnp.sum(r * r, axis=1, keepdims=True))
    delta = gs_ref[0, 0, 0] * (r / jnp.maximum(norm, _EPS))
    iota_col = jax.lax.broadcasted_iota(jnp.int32, (S, 1), 0)
    sel = (iota_col == li_ref[0, 0, 0]).astype(jnp.float32)
    out_ref[0] = x_ref[0] + sel * delta


def kernel(x, attention_mask, keys, values):
    B, S, D = x.shape
    NK = keys.shape[0]
    temperature = math.sqrt(math.log1p(float(NK))) / max(float(D), 1.0)
    eff_k = min(NK, max(2, int(math.ceil(
        math.sqrt(float(NK)) * math.log1p(float(NK))))))

    mask3 = attention_mask.astype(jnp.float32).reshape(B, 1, S)

    qn3, rms3, li3 = pl.pallas_call(
        _stage1_body,
        grid=(B,),
        in_specs=[
            pl.BlockSpec((1, S, D), lambda b: (b, 0, 0)),
            pl.BlockSpec((1, 1, S), lambda b: (b, 0, 0)),
        ],
        out_specs=[
            pl.BlockSpec((1, 1, D), lambda b: (b, 0, 0)),
            pl.BlockSpec((1, 1, 128), lambda b: (b, 0, 0)),
            pl.BlockSpec((1, 1, 128), lambda b: (b, 0, 0)),
        ],
        out_shape=[
            jax.ShapeDtypeStruct((B, 1, D), jnp.bfloat16),
            jax.ShapeDtypeStruct((B, 1, 128), jnp.float32),
            jax.ShapeDtypeStruct((B, 1, 128), jnp.int32),
        ],
    )(x, mask3)
    qn = qn3.reshape(B, D)
    rmsarr = rms3.reshape(B, 128)

    KB = 1024
    nkb = NK // KB
    kn, kn8, sims = pl.pallas_call(
        _stage2_body,
        grid=(nkb,),
        in_specs=[
            pl.BlockSpec((KB, D), lambda i: (i, 0)),
            pl.BlockSpec((B, D), lambda i: (0, 0)),
        ],
        out_specs=[
            pl.BlockSpec((KB, D), lambda i: (i, 0)),
            pl.BlockSpec((KB, D), lambda i: (i, 0)),
            pl.BlockSpec((B, KB), lambda i: (0, i)),
        ],
        out_shape=[
            jax.ShapeDtypeStruct((NK, D), jnp.bfloat16),
            jax.ShapeDtypeStruct((NK, D), jnp.float8_e4m3fn),
            jax.ShapeDtypeStruct((B, NK), jnp.float32),
        ],
    )(keys, qn)

    pairs = [(i, j) for i in range(nkb) for j in range(i, nkb)]
    t_total = len(pairs)
    il = jnp.asarray([p[0] for p in pairs], dtype=jnp.int32)
    jl = jnp.asarray([p[1] for p in pairs], dtype=jnp.int32)
    nmrow, nmcol = pl.pallas_call(
        functools.partial(_stage3_body, nkb=nkb, t_total=t_total, strip=256),
        grid_spec=pltpu.PrefetchScalarGridSpec(
            num_scalar_prefetch=2,
            grid=(t_total,),
            in_specs=[
                pl.BlockSpec((KB, D), lambda t, il_r, jl_r: (il_r[t], 0)),
                pl.BlockSpec((KB, D), lambda t, il_r, jl_r: (jl_r[t], 0)),
                pl.BlockSpec((KB, KB), lambda t, il_r, jl_r: (0, 0)),
            ],
            out_specs=[
                pl.BlockSpec((1, 1, KB), lambda t, il_r, jl_r: (il_r[t], 0, 0)),
                pl.BlockSpec((nkb, KB), lambda t, il_r, jl_r: (0, 0)),
            ],
            scratch_shapes=[
                pltpu.VMEM((KB, 128), jnp.float32),
                pltpu.VMEM((nkb, KB), jnp.float32),
            ],
        ),
        out_shape=[
            jax.ShapeDtypeStruct((nkb, 1, KB), jnp.float32),
            jax.ShapeDtypeStruct((nkb, KB), jnp.float32),
        ],
    )(il, jl, kn8, kn8, jnp.eye(KB, dtype=jnp.float32))

    nma = nmrow.reshape(NK // 128, 128)
    nmb = nmcol.reshape(NK // 128, 128)
    weights, gscale = pl.pallas_call(
        functools.partial(_stage4_body, nk=NK, d=D, eff_k=eff_k,
                          temperature=temperature),
        out_shape=[
            jax.ShapeDtypeStruct((B, NK), jnp.float32),
            jax.ShapeDtypeStruct((B, 128), jnp.float32),
        ],
    )(sims, nma, nmb, rmsarr)

    retrieved = pl.pallas_call(
        _stage5_body,
        grid=(nkb,),
        in_specs=[
            pl.BlockSpec((B, KB), lambda j: (0, j)),
            pl.BlockSpec((KB, D), lambda j: (j, 0)),
        ],
        out_specs=pl.BlockSpec((B, D), lambda j: (0, 0)),
        out_shape=jax.ShapeDtypeStruct((B, D), jnp.float32),
    )(weights, values)

    ret3 = retrieved.reshape(B, 1, D)
    gs3 = gscale.reshape(B, 1, 128)
    out = pl.pallas_call(
        _stage6_body,
        grid=(B,),
        in_specs=[
            pl.BlockSpec((1, S, D), lambda b: (b, 0, 0)),
            pl.BlockSpec((1, 1, D), lambda b: (b, 0, 0)),
            pl.BlockSpec((1, 1, 128), lambda b: (b, 0, 0)),
            pl.BlockSpec((1, 1, 128), lambda b: (b, 0, 0)),
        ],
        out_specs=pl.BlockSpec((1, S, D), lambda b: (b, 0, 0)),
        out_shape=jax.ShapeDtypeStruct((B, S, D), jnp.float32),
    )(x, ret3, gs3, li3)

    return out
